# baseline (speedup 1.0000x reference)
import functools

import jax
import jax.numpy as jnp
from jax import lax
from jax.experimental import pallas as pl
from jax.experimental.pallas import tpu as pltpu
from jax.experimental.pallas import tpu_sc as plsc

_N = 10000
_E = 320000
_F = 128
_K = 16

_NC = 2
_NS = 16
_NW = _NC * _NS

_EB = 80
_NBLK = _E // _EB
_BASE_BLOCKS = _NBLK // _NW
_EXTRA = _NBLK - _BASE_BLOCKS * _NW

_ROWS_PER_TILE = 624

_HIGH = lax.Precision.HIGHEST


def _node_body(x_ref, wd_ref, bd_ref, ws_ref, bs_ref, h_ref, s_ref):
    xa = jnp.maximum(x_ref[...], 0.0)
    h = lax.dot(xa, wd_ref[...], precision=_HIGH) + bd_ref[...]
    h_ref[...] = jnp.maximum(h, 0.0)
    s = lax.dot(xa, ws_ref[...], precision=_HIGH) + bs_ref[...]
    s_ref[...] = jnp.maximum(s, 0.0)


def _gate_body(attr_ref, wg_ref, g_ref):
    g_ref[...] = lax.dot(attr_ref[...], wg_ref[...], precision=_HIGH)


def _mlp_body(p0_ref, p1_ref, s_ref, x_ref, u_ref,
              w10, b10, w20, b20, w11, b11, w21, b21, w12, b12, w22, b22,
              wl, bl, out1_ref, out2_ref):
    msged = s_ref[...] + p0_ref[...] + p1_ref[...]
    out2_ref[...] = msged
    tmp = msged
    for (w1, b1, w2, b2) in ((w10, b10, w20, b20),
                             (w11, b11, w21, b21),
                             (w12, b12, w22, b22)):
        y = jnp.maximum(tmp, 0.0)
        y = jnp.maximum(lax.dot(y, w1[...], precision=_HIGH) + b1[...], 0.0)
        y = lax.dot(y, w2[...], precision=_HIGH) + b2[...]
        tmp = tmp + y
    v = jnp.maximum(tmp, 0.0)
    out1_ref[...] = (lax.dot(v, wl[...], precision=_HIGH) + bl[...]
                     + x_ref[...] * u_ref[...])


def _sc_edge_body(h_hbm, g_hbm, src_hbm, dst_hbm, out_hbm,
                  src_v, dst_v, h_v, g_v, z_v, acc,
                  sem_src, sem_dst, sem_g, sem_gat, sem_sct, sem_z):
    c = lax.axis_index("c")
    s = lax.axis_index("s")
    wid = c * _NS + s
    nblk = jnp.where(wid < _EXTRA, _BASE_BLOCKS + 1, _BASE_BLOCKS)

    def _base(t):
        return (wid + _NW * t) * _EB

    def _start_fetch(t, b):
        base = _base(t)
        pltpu.async_copy(src_hbm.at[pl.ds(base, _EB)], src_v.at[b],
                         sem_src.at[b])
        pltpu.async_copy(dst_hbm.at[pl.ds(base, _EB)], dst_v.at[b],
                         sem_dst.at[b])
        pltpu.async_copy(g_hbm.at[pl.ds(base, _EB)], g_v.at[b], sem_g.at[b])

    def _wait_src(t, b):
        pltpu.make_async_copy(src_hbm.at[pl.ds(_base(t), _EB)], src_v.at[b],
                              sem_src.at[b]).wait()

    def _wait_dst(t, b):
        pltpu.make_async_copy(dst_hbm.at[pl.ds(_base(t), _EB)], dst_v.at[b],
                              sem_dst.at[b]).wait()

    def _wait_g(t, b):
        pltpu.make_async_copy(g_hbm.at[pl.ds(_base(t), _EB)], g_v.at[b],
                              sem_g.at[b]).wait()

    def _start_gather(b):
        pltpu.async_copy(h_hbm.at[src_v.at[b]], h_v.at[b], sem_gat.at[b])

    def _wait_gather(b):
        pltpu.make_async_copy(h_hbm.at[src_v.at[b]], h_v.at[b],
                              sem_gat.at[b]).wait()

    def _start_scatter(b):
        pltpu.async_copy(h_v.at[b], acc.at[dst_v.at[b]], sem_sct.at[b],
                         add=True)

    def _wait_scatter(b):
        pltpu.make_async_copy(h_v.at[b], acc.at[dst_v.at[b]],
                              sem_sct.at[b]).wait()

    _start_fetch(0, 0)
    _wait_src(0, 0)
    _start_gather(0)

    zero16 = jnp.zeros((16,), jnp.float32)

    def _zfill(r, carry):
        for j in range(_F // 16):
            z_v[r, pl.ds(j * 16, 16)] = zero16
        return carry

    lax.fori_loop(0, 16, _zfill, 0)
    zbase = s * _ROWS_PER_TILE
    nz = jnp.where(s == _NS - 1, 40, 39)

    def _zero_step(t, carry):
        pltpu.async_copy(z_v, acc.at[pl.ds(zbase + 16 * t, 16)], sem_z)
        return carry

    lax.fori_loop(0, nz, _zero_step, 0)

    def _zero_wait(t, carry):
        pltpu.make_async_copy(z_v, acc.at[pl.ds(zbase, 16)], sem_z).wait()
        return carry

    lax.fori_loop(0, nz, _zero_wait, 0)
    plsc.subcore_barrier()

    def _edge_step(t, carry):
        b = lax.rem(t, 2)
        nb = 1 - b

        @pl.when(t >= 1)
        def _():
            _wait_scatter(nb)

        @pl.when(t + 1 < nblk)
        def _():
            _start_fetch(t + 1, nb)

        _wait_gather(b)
        _wait_g(t, b)

        def _mul_step(i, carry2):
            for j in range(_F // 16):
                sl = pl.ds(j * 16, 16)
                h_v[b, i, sl] = h_v[b, i, sl] * g_v[b, i, sl]
            return carry2

        lax.fori_loop(0, _EB, _mul_step, 0)
        _wait_dst(t, b)
        _start_scatter(b)

        @pl.when(t + 1 < nblk)
        def _():
            _wait_src(t + 1, nb)
            _start_gather(nb)

        return carry

    lax.fori_loop(0, nblk, _edge_step, 0)
    _wait_scatter(lax.rem(nblk - 1, 2))
    plsc.subcore_barrier()

    @pl.when(s < _NS - 1)
    def _():
        pltpu.sync_copy(acc.at[pl.ds(zbase, _ROWS_PER_TILE)],
                        out_hbm.at[c, pl.ds(zbase, _ROWS_PER_TILE)])

    @pl.when(s == _NS - 1)
    def _():
        last = (_NS - 1) * _ROWS_PER_TILE
        pltpu.sync_copy(acc.at[pl.ds(last, _N - last)],
                        out_hbm.at[c, pl.ds(last, _N - last)])


@functools.cache
def _sc_edge_kernel():
    return pl.kernel(
        _sc_edge_body,
        out_type=jax.ShapeDtypeStruct((_NC, _N, _F), jnp.float32),
        mesh=plsc.VectorSubcoreMesh(core_axis_name="c", subcore_axis_name="s",
                                    num_cores=_NC, num_subcores=_NS),
        scratch_types=[
            pltpu.VMEM((2, _EB), jnp.int32),
            pltpu.VMEM((2, _EB), jnp.int32),
            pltpu.VMEM((2, _EB, _F), jnp.float32),
            pltpu.VMEM((2, _EB, _F), jnp.float32),
            pltpu.VMEM((16, _F), jnp.float32),
            pltpu.VMEM_SHARED((_N, _F), jnp.float32),
            pltpu.SemaphoreType.DMA((2,)),
            pltpu.SemaphoreType.DMA((2,)),
            pltpu.SemaphoreType.DMA((2,)),
            pltpu.SemaphoreType.DMA((2,)),
            pltpu.SemaphoreType.DMA((2,)),
            pltpu.SemaphoreType.DMA,
        ],
    )


def _sc_edge(h, g, src, dst):
    return _sc_edge_kernel()(h, g, src, dst)


def kernel(x, edge_index, edge_attr, u, W_same, b_same, W_diff, b_diff, W_G,
           res0_W1, res0_b1, res0_W2, res0_b2,
           res1_W1, res1_b1, res1_W2, res1_b2,
           res2_W1, res2_b1, res2_W2, res2_b2,
           W_last, b_last):
    src = edge_index[0]
    dst = edge_index[1]

    node_br = 2000
    h, s = pl.pallas_call(
        _node_body,
        grid=(_N // node_br,),
        in_specs=[
            pl.BlockSpec((node_br, _F), lambda i: (i, 0)),
            pl.BlockSpec((_F, _F), lambda i: (0, 0)),
            pl.BlockSpec((1, _F), lambda i: (0, 0)),
            pl.BlockSpec((_F, _F), lambda i: (0, 0)),
            pl.BlockSpec((1, _F), lambda i: (0, 0)),
        ],
        out_specs=[
            pl.BlockSpec((node_br, _F), lambda i: (i, 0)),
            pl.BlockSpec((node_br, _F), lambda i: (i, 0)),
        ],
        out_shape=[
            jax.ShapeDtypeStruct((_N, _F), jnp.float32),
            jax.ShapeDtypeStruct((_N, _F), jnp.float32),
        ],
    )(x, W_diff.T, b_diff[None, :], W_same.T, b_same[None, :])

    gate_br = 2560
    g = pl.pallas_call(
        _gate_body,
        grid=(_E // gate_br,),
        in_specs=[
            pl.BlockSpec((gate_br, _K), lambda i: (i, 0)),
            pl.BlockSpec((_K, _F), lambda i: (0, 0)),
        ],
        out_specs=pl.BlockSpec((gate_br, _F), lambda i: (i, 0)),
        out_shape=jax.ShapeDtypeStruct((_E, _F), jnp.float32),
    )(edge_attr, W_G.T)

    partials = _sc_edge(h, g, src, dst)

    mlp_br = 2000
    row_spec = pl.BlockSpec((mlp_br, _F), lambda i: (i, 0))
    w_spec = pl.BlockSpec((_F, _F), lambda i: (0, 0))
    b_spec = pl.BlockSpec((1, _F), lambda i: (0, 0))
    out1, out2 = pl.pallas_call(
        _mlp_body,
        grid=(_N // mlp_br,),
        in_specs=[row_spec, row_spec, row_spec, row_spec, b_spec,
                  w_spec, b_spec, w_spec, b_spec,
                  w_spec, b_spec, w_spec, b_spec,
                  w_spec, b_spec, w_spec, b_spec,
                  w_spec, b_spec],
        out_specs=[row_spec, row_spec],
        out_shape=[
            jax.ShapeDtypeStruct((_N, _F), jnp.float32),
            jax.ShapeDtypeStruct((_N, _F), jnp.float32),
        ],
    )(partials[0], partials[1], s, x, u,
      res0_W1.T, res0_b1[None, :], res0_W2.T, res0_b2[None, :],
      res1_W1.T, res1_b1[None, :], res1_W2.T, res1_b2[None, :],
      res2_W1.T, res2_b1[None, :], res2_W2.T, res2_b2[None, :],
      W_last.T, b_last[None, :])
    return (out1, out2)


# speedup vs baseline: 1.1021x; 1.1021x over previous
import functools

import jax
import jax.numpy as jnp
from jax import lax
from jax.experimental import pallas as pl
from jax.experimental.pallas import tpu as pltpu
from jax.experimental.pallas import tpu_sc as plsc

_N = 10000
_E = 320000
_F = 128
_K = 16

_NC = 2
_NS = 16
_NW = _NC * _NS

_EB = 80
_NBLK = _E // _EB
_BASE_BLOCKS = _NBLK // _NW
_EXTRA = _NBLK - _BASE_BLOCKS * _NW

_ROWS_PER_TILE = 624

_HIGH = lax.Precision.HIGHEST


def _node_body(x_ref, wd_ref, bd_ref, ws_ref, bs_ref, h_ref, s_ref):
    xa = jnp.maximum(x_ref[...], 0.0)
    h = lax.dot(xa, wd_ref[...], precision=_HIGH) + bd_ref[...]
    h_ref[...] = jnp.maximum(h, 0.0)
    s = lax.dot(xa, ws_ref[...], precision=_HIGH) + bs_ref[...]
    s_ref[...] = jnp.maximum(s, 0.0)


def _gate_body(attr_ref, wg_ref, g_ref):
    g_ref[...] = lax.dot(attr_ref[...], wg_ref[...], precision=_HIGH)


def _mlp_body(p0_ref, p1_ref, s_ref, x_ref, u_ref,
              w10, b10, w20, b20, w11, b11, w21, b21, w12, b12, w22, b22,
              wl, bl, out1_ref, out2_ref):
    msged = s_ref[...] + p0_ref[...] + p1_ref[...]
    out2_ref[...] = msged
    tmp = msged
    for (w1, b1, w2, b2) in ((w10, b10, w20, b20),
                             (w11, b11, w21, b21),
                             (w12, b12, w22, b22)):
        y = jnp.maximum(tmp, 0.0)
        y = jnp.maximum(lax.dot(y, w1[...], precision=_HIGH) + b1[...], 0.0)
        y = lax.dot(y, w2[...], precision=_HIGH) + b2[...]
        tmp = tmp + y
    v = jnp.maximum(tmp, 0.0)
    out1_ref[...] = (lax.dot(v, wl[...], precision=_HIGH) + bl[...]
                     + x_ref[...] * u_ref[...])


def _sc_edge_body(h_hbm, g_hbm, src_hbm, dst_hbm, out_hbm,
                  src_v, dst_v, h_v, g_v, z_v, acc,
                  sem_src, sem_dst, sem_g, sem_gat, sem_sct, sem_z):
    c = lax.axis_index("c")
    s = lax.axis_index("s")
    wid = c * _NS + s
    nblk = jnp.where(wid < _EXTRA, _BASE_BLOCKS + 1, _BASE_BLOCKS)

    def _base(t):
        return (wid + _NW * t) * _EB

    def _start_fetch(t, b):
        base = _base(t)
        pltpu.async_copy(src_hbm.at[pl.ds(base, _EB)], src_v.at[b],
                         sem_src.at[b])
        pltpu.async_copy(dst_hbm.at[pl.ds(base, _EB)], dst_v.at[b],
                         sem_dst.at[b])
        pltpu.async_copy(g_hbm.at[pl.ds(base, _EB)], g_v.at[b], sem_g.at[b])

    def _wait_src(t, b):
        pltpu.make_async_copy(src_hbm.at[pl.ds(_base(t), _EB)], src_v.at[b],
                              sem_src.at[b]).wait()

    def _wait_dst(t, b):
        pltpu.make_async_copy(dst_hbm.at[pl.ds(_base(t), _EB)], dst_v.at[b],
                              sem_dst.at[b]).wait()

    def _wait_g(t, b):
        pltpu.make_async_copy(g_hbm.at[pl.ds(_base(t), _EB)], g_v.at[b],
                              sem_g.at[b]).wait()

    def _start_gather(b):
        pltpu.async_copy(h_hbm.at[src_v.at[b]], h_v.at[b], sem_gat.at[b])

    def _wait_gather(b):
        pltpu.make_async_copy(h_hbm.at[src_v.at[b]], h_v.at[b],
                              sem_gat.at[b]).wait()

    def _start_scatter(b):
        pltpu.async_copy(h_v.at[b], acc.at[dst_v.at[b]], sem_sct.at[b],
                         add=True)

    def _wait_scatter(b):
        pltpu.make_async_copy(h_v.at[b], acc.at[dst_v.at[b]],
                              sem_sct.at[b]).wait()

    _start_fetch(0, 0)
    _wait_src(0, 0)
    _start_gather(0)

    zero16 = jnp.zeros((16,), jnp.float32)

    def _zfill(r, carry):
        for j in range(_F // 16):
            z_v[r, pl.ds(j * 16, 16)] = zero16
        return carry

    lax.fori_loop(0, 16, _zfill, 0)
    zbase = s * _ROWS_PER_TILE
    nz = jnp.where(s == _NS - 1, 40, 39)

    def _zero_step(t, carry):
        pltpu.async_copy(z_v, acc.at[pl.ds(zbase + 16 * t, 16)], sem_z)
        return carry

    lax.fori_loop(0, nz, _zero_step, 0)

    def _zero_wait(t, carry):
        pltpu.make_async_copy(z_v, acc.at[pl.ds(zbase, 16)], sem_z).wait()
        return carry

    lax.fori_loop(0, nz, _zero_wait, 0)
    plsc.subcore_barrier()

    def _block(t, b, first, prefetch):
        if not first:
            _wait_scatter(1 - b)
        if prefetch:
            _start_fetch(t + 1, 1 - b)
        _wait_gather(b)
        _wait_g(t, b)
        hb = h_v.at[b]
        gb = g_v.at[b]

        def _mul_step(i, carry2):
            for j in range(_F // 16):
                sl = pl.ds(j * 16, 16)
                hb[i, sl] = hb[i, sl] * gb[i, sl]
            return carry2

        lax.fori_loop(0, _EB, _mul_step, 0, unroll=2)
        _wait_dst(t, b)
        _start_scatter(b)
        if prefetch:
            _wait_src(t + 1, 1 - b)
            _start_gather(1 - b)

    _block(0, 0, first=True, prefetch=True)

    def _pair_step(t2, carry):
        _block(2 * t2 + 1, 1, first=False, prefetch=True)
        _block(2 * t2 + 2, 0, first=False, prefetch=True)
        return carry

    lax.fori_loop(0, (_BASE_BLOCKS - 3) // 2, _pair_step, 0)
    _block(_BASE_BLOCKS - 2, 1, first=False, prefetch=True)
    _block(_BASE_BLOCKS - 1, 0, first=False, prefetch=False)
    _wait_scatter(0)
    plsc.subcore_barrier()

    @pl.when(s < _NS - 1)
    def _():
        pltpu.sync_copy(acc.at[pl.ds(zbase, _ROWS_PER_TILE)],
                        out_hbm.at[c, pl.ds(zbase, _ROWS_PER_TILE)])

    @pl.when(s == _NS - 1)
    def _():
        last = (_NS - 1) * _ROWS_PER_TILE
        pltpu.sync_copy(acc.at[pl.ds(last, _N - last)],
                        out_hbm.at[c, pl.ds(last, _N - last)])


@functools.cache
def _sc_edge_kernel():
    return pl.kernel(
        _sc_edge_body,
        out_type=jax.ShapeDtypeStruct((_NC, _N, _F), jnp.float32),
        mesh=plsc.VectorSubcoreMesh(core_axis_name="c", subcore_axis_name="s",
                                    num_cores=_NC, num_subcores=_NS),
        scratch_types=[
            pltpu.VMEM((2, _EB), jnp.int32),
            pltpu.VMEM((2, _EB), jnp.int32),
            pltpu.VMEM((2, _EB, _F), jnp.float32),
            pltpu.VMEM((2, _EB, _F), jnp.float32),
            pltpu.VMEM((16, _F), jnp.float32),
            pltpu.VMEM_SHARED((_N, _F), jnp.float32),
            pltpu.SemaphoreType.DMA((2,)),
            pltpu.SemaphoreType.DMA((2,)),
            pltpu.SemaphoreType.DMA((2,)),
            pltpu.SemaphoreType.DMA((2,)),
            pltpu.SemaphoreType.DMA((2,)),
            pltpu.SemaphoreType.DMA,
        ],
    )


def _sc_edge(h, g, src, dst):
    return _sc_edge_kernel()(h, g, src, dst)


def kernel(x, edge_index, edge_attr, u, W_same, b_same, W_diff, b_diff, W_G,
           res0_W1, res0_b1, res0_W2, res0_b2,
           res1_W1, res1_b1, res1_W2, res1_b2,
           res2_W1, res2_b1, res2_W2, res2_b2,
           W_last, b_last):
    src = edge_index[0]
    dst = edge_index[1]

    node_br = 2000
    h, s = pl.pallas_call(
        _node_body,
        grid=(_N // node_br,),
        in_specs=[
            pl.BlockSpec((node_br, _F), lambda i: (i, 0)),
            pl.BlockSpec((_F, _F), lambda i: (0, 0)),
            pl.BlockSpec((1, _F), lambda i: (0, 0)),
            pl.BlockSpec((_F, _F), lambda i: (0, 0)),
            pl.BlockSpec((1, _F), lambda i: (0, 0)),
        ],
        out_specs=[
            pl.BlockSpec((node_br, _F), lambda i: (i, 0)),
            pl.BlockSpec((node_br, _F), lambda i: (i, 0)),
        ],
        out_shape=[
            jax.ShapeDtypeStruct((_N, _F), jnp.float32),
            jax.ShapeDtypeStruct((_N, _F), jnp.float32),
        ],
    )(x, W_diff.T, b_diff[None, :], W_same.T, b_same[None, :])

    gate_br = 2560
    g = pl.pallas_call(
        _gate_body,
        grid=(_E // gate_br,),
        in_specs=[
            pl.BlockSpec((gate_br, _K), lambda i: (i, 0)),
            pl.BlockSpec((_K, _F), lambda i: (0, 0)),
        ],
        out_specs=pl.BlockSpec((gate_br, _F), lambda i: (i, 0)),
        out_shape=jax.ShapeDtypeStruct((_E, _F), jnp.float32),
    )(edge_attr, W_G.T)

    partials = _sc_edge(h, g, src, dst)

    mlp_br = 2000
    row_spec = pl.BlockSpec((mlp_br, _F), lambda i: (i, 0))
    w_spec = pl.BlockSpec((_F, _F), lambda i: (0, 0))
    b_spec = pl.BlockSpec((1, _F), lambda i: (0, 0))
    out1, out2 = pl.pallas_call(
        _mlp_body,
        grid=(_N // mlp_br,),
        in_specs=[row_spec, row_spec, row_spec, row_spec, b_spec,
                  w_spec, b_spec, w_spec, b_spec,
                  w_spec, b_spec, w_spec, b_spec,
                  w_spec, b_spec, w_spec, b_spec,
                  w_spec, b_spec],
        out_specs=[row_spec, row_spec],
        out_shape=[
            jax.ShapeDtypeStruct((_N, _F), jnp.float32),
            jax.ShapeDtypeStruct((_N, _F), jnp.float32),
        ],
    )(partials[0], partials[1], s, x, u,
      res0_W1.T, res0_b1[None, :], res0_W2.T, res0_b2[None, :],
      res1_W1.T, res1_b1[None, :], res1_W2.T, res1_b2[None, :],
      res2_W1.T, res2_b1[None, :], res2_W2.T, res2_b2[None, :],
      W_last.T, b_last[None, :])
    return (out1, out2)


# speedup vs baseline: 1.5755x; 1.4296x over previous
import functools

import jax
import jax.numpy as jnp
from jax import lax
from jax.experimental import pallas as pl
from jax.experimental.pallas import tpu as pltpu
from jax.experimental.pallas import tpu_sc as plsc

_N = 10000
_E = 320000
_F = 128
_K = 16

_NC = 2
_NS = 16
_NW = _NC * _NS

_EB = 80
_NBLK = _E // _EB
_BASE_BLOCKS = _NBLK // _NW
_EXTRA = _NBLK - _BASE_BLOCKS * _NW

_ROWS_PER_TILE = 624

_HIGH = lax.Precision.HIGHEST
_DIAG_MUL = True
_DIAG_SCATTER = True


def _node_body(x_ref, wd_ref, bd_ref, ws_ref, bs_ref, h_ref, s_ref):
    xa = jnp.maximum(x_ref[...], 0.0)
    h = lax.dot(xa, wd_ref[...], precision=_HIGH) + bd_ref[...]
    h_ref[...] = jnp.maximum(h, 0.0)
    s = lax.dot(xa, ws_ref[...], precision=_HIGH) + bs_ref[...]
    s_ref[...] = jnp.maximum(s, 0.0)


def _gate_body(attr_ref, wg_ref, g_ref):
    g_ref[...] = lax.dot(attr_ref[...], wg_ref[...], precision=_HIGH)


def _mlp_body(p0_ref, p1_ref, s_ref, x_ref, u_ref,
              w10, b10, w20, b20, w11, b11, w21, b21, w12, b12, w22, b22,
              wl, bl, out1_ref, out2_ref):
    msged = s_ref[...] + p0_ref[...] + p1_ref[...]
    out2_ref[...] = msged
    tmp = msged
    for (w1, b1, w2, b2) in ((w10, b10, w20, b20),
                             (w11, b11, w21, b21),
                             (w12, b12, w22, b22)):
        y = jnp.maximum(tmp, 0.0)
        y = jnp.maximum(lax.dot(y, w1[...], precision=_HIGH) + b1[...], 0.0)
        y = lax.dot(y, w2[...], precision=_HIGH) + b2[...]
        tmp = tmp + y
    v = jnp.maximum(tmp, 0.0)
    out1_ref[...] = (lax.dot(v, wl[...], precision=_HIGH) + bl[...]
                     + x_ref[...] * u_ref[...])


def _sc_edge_body(h_hbm, g_hbm, src_hbm, dst_hbm, out_hbm,
                  src_v, dst_v, h_v, g_v, z_v, acc,
                  sem_src, sem_dst, sem_g, sem_gat, sem_sct, sem_z):
    c = lax.axis_index("c")
    s = lax.axis_index("s")
    wid = c * _NS + s
    nblk = jnp.where(wid < _EXTRA, _BASE_BLOCKS + 1, _BASE_BLOCKS)

    def _base(t):
        return (wid + _NW * t) * _EB

    def _start_fetch(t, b):
        base = _base(t)
        pltpu.async_copy(src_hbm.at[pl.ds(base, _EB)], src_v.at[b],
                         sem_src.at[b])
        pltpu.async_copy(dst_hbm.at[pl.ds(base, _EB)], dst_v.at[b],
                         sem_dst.at[b])
        pltpu.async_copy(g_hbm.at[pl.ds(base, _EB)], g_v.at[b], sem_g.at[b])

    def _wait_src(t, b):
        pltpu.make_async_copy(src_hbm.at[pl.ds(_base(t), _EB)], src_v.at[b],
                              sem_src.at[b]).wait()

    def _wait_dst(t, b):
        pltpu.make_async_copy(dst_hbm.at[pl.ds(_base(t), _EB)], dst_v.at[b],
                              sem_dst.at[b]).wait()

    def _wait_g(t, b):
        pltpu.make_async_copy(g_hbm.at[pl.ds(_base(t), _EB)], g_v.at[b],
                              sem_g.at[b]).wait()

    def _start_gather(b):
        pltpu.async_copy(h_hbm.at[src_v.at[b]], h_v.at[b], sem_gat.at[b])

    def _wait_gather(b):
        pltpu.make_async_copy(h_hbm.at[src_v.at[b]], h_v.at[b],
                              sem_gat.at[b]).wait()

    def _start_scatter(b):
        if _DIAG_SCATTER:
            pltpu.async_copy(h_v.at[b], acc.at[dst_v.at[b]], sem_sct.at[b],
                             add=True)

    def _wait_scatter(b):
        if _DIAG_SCATTER:
            pltpu.make_async_copy(h_v.at[b], acc.at[dst_v.at[b]],
                                  sem_sct.at[b]).wait()

    _start_fetch(0, 0)
    _wait_src(0, 0)
    _start_gather(0)

    zero16 = jnp.zeros((16,), jnp.float32)

    def _zfill(r, carry):
        for j in range(_F // 16):
            z_v[r, pl.ds(j * 16, 16)] = zero16
        return carry

    lax.fori_loop(0, 16, _zfill, 0)
    zbase = s * _ROWS_PER_TILE
    nz = jnp.where(s == _NS - 1, 40, 39)

    def _zero_step(t, carry):
        pltpu.async_copy(z_v, acc.at[pl.ds(zbase + 16 * t, 16)], sem_z)
        return carry

    lax.fori_loop(0, nz, _zero_step, 0)

    def _zero_wait(t, carry):
        pltpu.make_async_copy(z_v, acc.at[pl.ds(zbase, 16)], sem_z).wait()
        return carry

    lax.fori_loop(0, nz, _zero_wait, 0)
    plsc.subcore_barrier()

    def _block(t, b, first, prefetch):
        if not first:
            _wait_scatter(1 - b)
        if prefetch:
            _start_fetch(t + 1, 1 - b)
        _wait_gather(b)
        _wait_g(t, b)
        hb = h_v.at[b]
        gb = g_v.at[b]

        if _DIAG_MUL:
            @plsc.parallel_loop(0, _EB, unroll=2)
            def _mul_step(i):
                for j in range(_F // 16):
                    sl = pl.ds(j * 16, 16)
                    hb[i, sl] = hb[i, sl] * gb[i, sl]
        _wait_dst(t, b)
        _start_scatter(b)
        if prefetch:
            _wait_src(t + 1, 1 - b)
            _start_gather(1 - b)

    _block(0, 0, first=True, prefetch=True)

    def _pair_step(t2, carry):
        _block(2 * t2 + 1, 1, first=False, prefetch=True)
        _block(2 * t2 + 2, 0, first=False, prefetch=True)
        return carry

    lax.fori_loop(0, (_BASE_BLOCKS - 3) // 2, _pair_step, 0)
    _block(_BASE_BLOCKS - 2, 1, first=False, prefetch=True)
    _block(_BASE_BLOCKS - 1, 0, first=False, prefetch=False)
    _wait_scatter(0)
    plsc.subcore_barrier()

    @pl.when(s < _NS - 1)
    def _():
        pltpu.sync_copy(acc.at[pl.ds(zbase, _ROWS_PER_TILE)],
                        out_hbm.at[c, pl.ds(zbase, _ROWS_PER_TILE)])

    @pl.when(s == _NS - 1)
    def _():
        last = (_NS - 1) * _ROWS_PER_TILE
        pltpu.sync_copy(acc.at[pl.ds(last, _N - last)],
                        out_hbm.at[c, pl.ds(last, _N - last)])


@functools.cache
def _sc_edge_kernel():
    return pl.kernel(
        _sc_edge_body,
        out_type=jax.ShapeDtypeStruct((_NC, _N, _F), jnp.float32),
        mesh=plsc.VectorSubcoreMesh(core_axis_name="c", subcore_axis_name="s",
                                    num_cores=_NC, num_subcores=_NS),
        scratch_types=[
            pltpu.VMEM((2, _EB), jnp.int32),
            pltpu.VMEM((2, _EB), jnp.int32),
            pltpu.VMEM((2, _EB, _F), jnp.float32),
            pltpu.VMEM((2, _EB, _F), jnp.float32),
            pltpu.VMEM((16, _F), jnp.float32),
            pltpu.VMEM_SHARED((_N, _F), jnp.float32),
            pltpu.SemaphoreType.DMA((2,)),
            pltpu.SemaphoreType.DMA((2,)),
            pltpu.SemaphoreType.DMA((2,)),
            pltpu.SemaphoreType.DMA((2,)),
            pltpu.SemaphoreType.DMA((2,)),
            pltpu.SemaphoreType.DMA,
        ],
    )


def _sc_edge(h, g, src, dst):
    return _sc_edge_kernel()(h, g, src, dst)


def kernel(x, edge_index, edge_attr, u, W_same, b_same, W_diff, b_diff, W_G,
           res0_W1, res0_b1, res0_W2, res0_b2,
           res1_W1, res1_b1, res1_W2, res1_b2,
           res2_W1, res2_b1, res2_W2, res2_b2,
           W_last, b_last):
    src = edge_index[0]
    dst = edge_index[1]

    node_br = 2000
    h, s = pl.pallas_call(
        _node_body,
        grid=(_N // node_br,),
        in_specs=[
            pl.BlockSpec((node_br, _F), lambda i: (i, 0)),
            pl.BlockSpec((_F, _F), lambda i: (0, 0)),
            pl.BlockSpec((1, _F), lambda i: (0, 0)),
            pl.BlockSpec((_F, _F), lambda i: (0, 0)),
            pl.BlockSpec((1, _F), lambda i: (0, 0)),
        ],
        out_specs=[
            pl.BlockSpec((node_br, _F), lambda i: (i, 0)),
            pl.BlockSpec((node_br, _F), lambda i: (i, 0)),
        ],
        out_shape=[
            jax.ShapeDtypeStruct((_N, _F), jnp.float32),
            jax.ShapeDtypeStruct((_N, _F), jnp.float32),
        ],
    )(x, W_diff.T, b_diff[None, :], W_same.T, b_same[None, :])

    gate_br = 2560
    g = pl.pallas_call(
        _gate_body,
        grid=(_E // gate_br,),
        in_specs=[
            pl.BlockSpec((gate_br, _K), lambda i: (i, 0)),
            pl.BlockSpec((_K, _F), lambda i: (0, 0)),
        ],
        out_specs=pl.BlockSpec((gate_br, _F), lambda i: (i, 0)),
        out_shape=jax.ShapeDtypeStruct((_E, _F), jnp.float32),
    )(edge_attr, W_G.T)

    partials = _sc_edge(h, g, src, dst)

    mlp_br = 2000
    row_spec = pl.BlockSpec((mlp_br, _F), lambda i: (i, 0))
    w_spec = pl.BlockSpec((_F, _F), lambda i: (0, 0))
    b_spec = pl.BlockSpec((1, _F), lambda i: (0, 0))
    out1, out2 = pl.pallas_call(
        _mlp_body,
        grid=(_N // mlp_br,),
        in_specs=[row_spec, row_spec, row_spec, row_spec, b_spec,
                  w_spec, b_spec, w_spec, b_spec,
                  w_spec, b_spec, w_spec, b_spec,
                  w_spec, b_spec, w_spec, b_spec,
                  w_spec, b_spec],
        out_specs=[row_spec, row_spec],
        out_shape=[
            jax.ShapeDtypeStruct((_N, _F), jnp.float32),
            jax.ShapeDtypeStruct((_N, _F), jnp.float32),
        ],
    )(partials[0], partials[1], s, x, u,
      res0_W1.T, res0_b1[None, :], res0_W2.T, res0_b2[None, :],
      res1_W1.T, res1_b1[None, :], res1_W2.T, res1_b2[None, :],
      res2_W1.T, res2_b1[None, :], res2_W2.T, res2_b2[None, :],
      W_last.T, b_last[None, :])
    return (out1, out2)


# speedup vs baseline: 1.7054x; 1.0825x over previous
import functools

import jax
import jax.numpy as jnp
from jax import lax
from jax.experimental import pallas as pl
from jax.experimental.pallas import tpu as pltpu
from jax.experimental.pallas import tpu_sc as plsc

_N = 10000
_E = 320000
_F = 128
_K = 16

_NC = 2
_NS = 16
_NW = _NC * _NS

_EB = 80
_NBLK = _E // _EB
_BASE_BLOCKS = _NBLK // _NW
_EXTRA = _NBLK - _BASE_BLOCKS * _NW

_ROWS_PER_TILE = 624

_HIGH = lax.Precision.HIGHEST
_DIAG_MUL = True
_DIAG_SCATTER = True


def _node_body(x_ref, wd_ref, bd_ref, ws_ref, bs_ref, h_ref, s_ref):
    xa = jnp.maximum(x_ref[...], 0.0)
    h = lax.dot(xa, wd_ref[...], precision=_HIGH) + bd_ref[...]
    h_ref[...] = jnp.maximum(h, 0.0)
    s = lax.dot(xa, ws_ref[...], precision=_HIGH) + bs_ref[...]
    s_ref[...] = jnp.maximum(s, 0.0)


def _gate_body(attr_ref, wg_ref, g_ref):
    g_ref[...] = lax.dot(attr_ref[...], wg_ref[...],
                         precision=lax.Precision.DEFAULT)


def _mlp_body(p0_ref, p1_ref, s_ref, x_ref, u_ref,
              w10, b10, w20, b20, w11, b11, w21, b21, w12, b12, w22, b22,
              wl, bl, out1_ref, out2_ref):
    msged = s_ref[...] + p0_ref[...] + p1_ref[...]
    out2_ref[...] = msged
    tmp = msged
    for (w1, b1, w2, b2) in ((w10, b10, w20, b20),
                             (w11, b11, w21, b21),
                             (w12, b12, w22, b22)):
        y = jnp.maximum(tmp, 0.0)
        y = jnp.maximum(lax.dot(y, w1[...], precision=_HIGH) + b1[...], 0.0)
        y = lax.dot(y, w2[...], precision=_HIGH) + b2[...]
        tmp = tmp + y
    v = jnp.maximum(tmp, 0.0)
    out1_ref[...] = (lax.dot(v, wl[...], precision=_HIGH) + bl[...]
                     + x_ref[...] * u_ref[...])


def _sc_edge_body(h_hbm, g_hbm, src_hbm, dst_hbm, out_hbm,
                  src_v, dst_v, h_v, g_v, z_v, acc,
                  sem_src, sem_dst, sem_g, sem_gat, sem_sct, sem_z):
    c = lax.axis_index("c")
    s = lax.axis_index("s")
    wid = c * _NS + s
    nblk = jnp.where(wid < _EXTRA, _BASE_BLOCKS + 1, _BASE_BLOCKS)

    def _base(t):
        return (wid + _NW * t) * _EB

    def _start_fetch(t, b):
        base = _base(t)
        pltpu.async_copy(src_hbm.at[pl.ds(base, _EB)], src_v.at[b],
                         sem_src.at[b])
        pltpu.async_copy(dst_hbm.at[pl.ds(base, _EB)], dst_v.at[b],
                         sem_dst.at[b])
        pltpu.async_copy(g_hbm.at[pl.ds(base, _EB)], g_v.at[b], sem_g.at[b])

    def _wait_src(t, b):
        pltpu.make_async_copy(src_hbm.at[pl.ds(_base(t), _EB)], src_v.at[b],
                              sem_src.at[b]).wait()

    def _wait_dst(t, b):
        pltpu.make_async_copy(dst_hbm.at[pl.ds(_base(t), _EB)], dst_v.at[b],
                              sem_dst.at[b]).wait()

    def _wait_g(t, b):
        pltpu.make_async_copy(g_hbm.at[pl.ds(_base(t), _EB)], g_v.at[b],
                              sem_g.at[b]).wait()

    def _start_gather(b):
        pltpu.async_copy(h_hbm.at[src_v.at[b]], h_v.at[b], sem_gat.at[b])

    def _wait_gather(b):
        pltpu.make_async_copy(h_hbm.at[src_v.at[b]], h_v.at[b],
                              sem_gat.at[b]).wait()

    def _start_scatter(b):
        if _DIAG_SCATTER:
            pltpu.async_copy(h_v.at[b], acc.at[dst_v.at[b]], sem_sct.at[b],
                             add=True)

    def _wait_scatter(b):
        if _DIAG_SCATTER:
            pltpu.make_async_copy(h_v.at[b], acc.at[dst_v.at[b]],
                                  sem_sct.at[b]).wait()

    _start_fetch(0, 0)
    _wait_src(0, 0)
    _start_gather(0)

    zero16 = jnp.zeros((16,), jnp.float32)

    def _zfill(r, carry):
        for j in range(_F // 16):
            z_v[r, pl.ds(j * 16, 16)] = zero16
        return carry

    lax.fori_loop(0, 16, _zfill, 0)
    zbase = s * _ROWS_PER_TILE
    nz = jnp.where(s == _NS - 1, 40, 39)

    def _zero_step(t, carry):
        pltpu.async_copy(z_v, acc.at[pl.ds(zbase + 16 * t, 16)], sem_z)
        return carry

    lax.fori_loop(0, nz, _zero_step, 0)

    def _zero_wait(t, carry):
        pltpu.make_async_copy(z_v, acc.at[pl.ds(zbase, 16)], sem_z).wait()
        return carry

    lax.fori_loop(0, nz, _zero_wait, 0)
    plsc.subcore_barrier()

    def _block(t, b, first, prefetch):
        if not first:
            _wait_scatter(1 - b)
        if prefetch:
            _start_fetch(t + 1, 1 - b)
        _wait_gather(b)
        _wait_g(t, b)
        hb = h_v.at[b]
        gb = g_v.at[b]

        if _DIAG_MUL:
            @plsc.parallel_loop(0, _EB, unroll=4)
            def _mul_step(i):
                for j in range(_F // 16):
                    sl = pl.ds(j * 16, 16)
                    hb[i, sl] = hb[i, sl] * gb[i, sl]
        _wait_dst(t, b)
        _start_scatter(b)
        if prefetch:
            _wait_src(t + 1, 1 - b)
            _start_gather(1 - b)

    _block(0, 0, first=True, prefetch=True)

    def _pair_step(t2, carry):
        _block(2 * t2 + 1, 1, first=False, prefetch=True)
        _block(2 * t2 + 2, 0, first=False, prefetch=True)
        return carry

    lax.fori_loop(0, (_BASE_BLOCKS - 3) // 2, _pair_step, 0)
    _block(_BASE_BLOCKS - 2, 1, first=False, prefetch=True)
    _block(_BASE_BLOCKS - 1, 0, first=False, prefetch=False)
    _wait_scatter(0)
    plsc.subcore_barrier()

    @pl.when(s < _NS - 1)
    def _():
        pltpu.sync_copy(acc.at[pl.ds(zbase, _ROWS_PER_TILE)],
                        out_hbm.at[c, pl.ds(zbase, _ROWS_PER_TILE)])

    @pl.when(s == _NS - 1)
    def _():
        last = (_NS - 1) * _ROWS_PER_TILE
        pltpu.sync_copy(acc.at[pl.ds(last, _N - last)],
                        out_hbm.at[c, pl.ds(last, _N - last)])


@functools.cache
def _sc_edge_kernel():
    return pl.kernel(
        _sc_edge_body,
        out_type=jax.ShapeDtypeStruct((_NC, _N, _F), jnp.float32),
        mesh=plsc.VectorSubcoreMesh(core_axis_name="c", subcore_axis_name="s",
                                    num_cores=_NC, num_subcores=_NS),
        scratch_types=[
            pltpu.VMEM((2, _EB), jnp.int32),
            pltpu.VMEM((2, _EB), jnp.int32),
            pltpu.VMEM((2, _EB, _F), jnp.float32),
            pltpu.VMEM((2, _EB, _F), jnp.float32),
            pltpu.VMEM((16, _F), jnp.float32),
            pltpu.VMEM_SHARED((_N, _F), jnp.float32),
            pltpu.SemaphoreType.DMA((2,)),
            pltpu.SemaphoreType.DMA((2,)),
            pltpu.SemaphoreType.DMA((2,)),
            pltpu.SemaphoreType.DMA((2,)),
            pltpu.SemaphoreType.DMA((2,)),
            pltpu.SemaphoreType.DMA,
        ],
    )


def _sc_edge(h, g, src, dst):
    return _sc_edge_kernel()(h, g, src, dst)


def kernel(x, edge_index, edge_attr, u, W_same, b_same, W_diff, b_diff, W_G,
           res0_W1, res0_b1, res0_W2, res0_b2,
           res1_W1, res1_b1, res1_W2, res1_b2,
           res2_W1, res2_b1, res2_W2, res2_b2,
           W_last, b_last):
    src = edge_index[0]
    dst = edge_index[1]

    node_br = 2000
    h, s = pl.pallas_call(
        _node_body,
        grid=(_N // node_br,),
        in_specs=[
            pl.BlockSpec((node_br, _F), lambda i: (i, 0)),
            pl.BlockSpec((_F, _F), lambda i: (0, 0)),
            pl.BlockSpec((1, _F), lambda i: (0, 0)),
            pl.BlockSpec((_F, _F), lambda i: (0, 0)),
            pl.BlockSpec((1, _F), lambda i: (0, 0)),
        ],
        out_specs=[
            pl.BlockSpec((node_br, _F), lambda i: (i, 0)),
            pl.BlockSpec((node_br, _F), lambda i: (i, 0)),
        ],
        out_shape=[
            jax.ShapeDtypeStruct((_N, _F), jnp.float32),
            jax.ShapeDtypeStruct((_N, _F), jnp.float32),
        ],
    )(x, W_diff.T, b_diff[None, :], W_same.T, b_same[None, :])

    gate_br = 2560
    g = pl.pallas_call(
        _gate_body,
        grid=(_E // gate_br,),
        in_specs=[
            pl.BlockSpec((gate_br, _K), lambda i: (i, 0)),
            pl.BlockSpec((_K, _F), lambda i: (0, 0)),
        ],
        out_specs=pl.BlockSpec((gate_br, _F), lambda i: (i, 0)),
        out_shape=jax.ShapeDtypeStruct((_E, _F), jnp.float32),
    )(edge_attr, W_G.T)

    partials = _sc_edge(h, g, src, dst)

    mlp_br = 2000
    row_spec = pl.BlockSpec((mlp_br, _F), lambda i: (i, 0))
    w_spec = pl.BlockSpec((_F, _F), lambda i: (0, 0))
    b_spec = pl.BlockSpec((1, _F), lambda i: (0, 0))
    out1, out2 = pl.pallas_call(
        _mlp_body,
        grid=(_N // mlp_br,),
        in_specs=[row_spec, row_spec, row_spec, row_spec, b_spec,
                  w_spec, b_spec, w_spec, b_spec,
                  w_spec, b_spec, w_spec, b_spec,
                  w_spec, b_spec, w_spec, b_spec,
                  w_spec, b_spec],
        out_specs=[row_spec, row_spec],
        out_shape=[
            jax.ShapeDtypeStruct((_N, _F), jnp.float32),
            jax.ShapeDtypeStruct((_N, _F), jnp.float32),
        ],
    )(partials[0], partials[1], s, x, u,
      res0_W1.T, res0_b1[None, :], res0_W2.T, res0_b2[None, :],
      res1_W1.T, res1_b1[None, :], res1_W2.T, res1_b2[None, :],
      res2_W1.T, res2_b1[None, :], res2_W2.T, res2_b2[None, :],
      W_last.T, b_last[None, :])
    return (out1, out2)


# speedup vs baseline: 2.1078x; 1.2360x over previous
import functools

import jax
import jax.numpy as jnp
from jax import lax
from jax.experimental import pallas as pl
from jax.experimental.pallas import tpu as pltpu
from jax.experimental.pallas import tpu_sc as plsc

_N = 10000
_E = 320000
_F = 128
_K = 16

_NC = 2
_NS = 16
_NW = _NC * _NS

_EB = 80
_NBLK = _E // _EB
_BASE_BLOCKS = _NBLK // _NW
_EXTRA = _NBLK - _BASE_BLOCKS * _NW

_ROWS_PER_TILE = 624

def _dotT(a, w):
    return lax.dot_general(a, w, dimension_numbers=(((1,), (1,)), ((), ())),
                           precision=lax.Precision.DEFAULT)


def _pre_body(x_ref, wd_ref, bd_ref, ws_ref, bs_ref, attr_ref, wg_ref,
              h_ref, s_ref, g_ref):
    xa = jnp.maximum(x_ref[...], 0.0)
    h_ref[...] = jnp.maximum(_dotT(xa, wd_ref[...]) + bd_ref[...], 0.0)
    s_ref[...] = jnp.maximum(_dotT(xa, ws_ref[...]) + bs_ref[...], 0.0)
    g_ref[...] = _dotT(attr_ref[...], wg_ref[...])


def _mlp_body(p_ref, s_ref, x_ref, u_ref,
              w10, b10, w20, b20, w11, b11, w21, b21, w12, b12, w22, b22,
              wl, bl, out1_ref, out2_ref):
    msged = s_ref[...] + p_ref[0] + p_ref[1]
    out2_ref[...] = msged
    tmp = msged
    for (w1, b1, w2, b2) in ((w10, b10, w20, b20),
                             (w11, b11, w21, b21),
                             (w12, b12, w22, b22)):
        y = jnp.maximum(tmp, 0.0)
        y = jnp.maximum(_dotT(y, w1[...]) + b1[...], 0.0)
        y = _dotT(y, w2[...]) + b2[...]
        tmp = tmp + y
    v = jnp.maximum(tmp, 0.0)
    out1_ref[...] = _dotT(v, wl[...]) + bl[...] + x_ref[...] * u_ref[...]


def _sc_edge_body(h_hbm, g_hbm, ei_hbm, out_hbm,
                  src_v, dst_v, h_v, g_v, z_v, acc,
                  sem_src, sem_dst, sem_g, sem_gat, sem_sct, sem_z):
    c = lax.axis_index("c")
    s = lax.axis_index("s")
    wid = c * _NS + s

    def _base(t):
        return (wid + _NW * t) * _EB

    def _start_fetch(t, b):
        base = _base(t)
        pltpu.async_copy(ei_hbm.at[pl.ds(base, _EB)], src_v.at[b],
                         sem_src.at[b])
        pltpu.async_copy(ei_hbm.at[pl.ds(_E + base, _EB)], dst_v.at[b],
                         sem_dst.at[b])
        pltpu.async_copy(g_hbm.at[pl.ds(base, _EB)], g_v.at[b], sem_g.at[b])

    def _wait_src(t, b):
        pltpu.make_async_copy(ei_hbm.at[pl.ds(_base(t), _EB)], src_v.at[b],
                              sem_src.at[b]).wait()

    def _wait_dst(t, b):
        pltpu.make_async_copy(ei_hbm.at[pl.ds(_E + _base(t), _EB)], dst_v.at[b],
                              sem_dst.at[b]).wait()

    def _wait_g(t, b):
        pltpu.make_async_copy(g_hbm.at[pl.ds(_base(t), _EB)], g_v.at[b],
                              sem_g.at[b]).wait()

    def _start_gather(b):
        pltpu.async_copy(h_hbm.at[src_v.at[b]], h_v.at[b], sem_gat.at[b])

    def _wait_gather(b):
        pltpu.make_async_copy(h_hbm.at[src_v.at[b]], h_v.at[b],
                              sem_gat.at[b]).wait()

    def _start_scatter(b):
        pltpu.async_copy(h_v.at[b], acc.at[dst_v.at[b]], sem_sct.at[b],
                         add=True)

    def _wait_scatter(b):
        pltpu.make_async_copy(h_v.at[b], acc.at[dst_v.at[b]],
                              sem_sct.at[b]).wait()

    _start_fetch(0, 0)
    _wait_src(0, 0)
    _start_gather(0)

    zero16 = jnp.zeros((16,), jnp.float32)

    def _zfill(r, carry):
        for j in range(_F // 16):
            z_v[r, pl.ds(j * 16, 16)] = zero16
        return carry

    lax.fori_loop(0, 16, _zfill, 0)
    zbase = s * _ROWS_PER_TILE
    nz = jnp.where(s == _NS - 1, 40, 39)

    def _zero_step(t, carry):
        pltpu.async_copy(z_v, acc.at[pl.ds(zbase + 16 * t, 16)], sem_z)
        return carry

    lax.fori_loop(0, nz, _zero_step, 0)

    def _zero_wait(t, carry):
        pltpu.make_async_copy(z_v, acc.at[pl.ds(zbase, 16)], sem_z).wait()
        return carry

    lax.fori_loop(0, nz, _zero_wait, 0)
    plsc.subcore_barrier()

    def _block(t, b, first, prefetch):
        if not first:
            _wait_scatter(1 - b)
        if prefetch:
            _start_fetch(t + 1, 1 - b)
        _wait_gather(b)
        _wait_g(t, b)
        hb = h_v.at[b]
        gb = g_v.at[b]

        @plsc.parallel_loop(0, _EB, unroll=4)
        def _mul_step(i):
            for j in range(_F // 16):
                sl = pl.ds(j * 16, 16)
                hb[i, sl] = hb[i, sl] * gb[i, sl]

        _wait_dst(t, b)
        _start_scatter(b)
        if prefetch:
            _wait_src(t + 1, 1 - b)
            _start_gather(1 - b)

    _block(0, 0, first=True, prefetch=True)

    def _pair_step(t2, carry):
        _block(2 * t2 + 1, 1, first=False, prefetch=True)
        _block(2 * t2 + 2, 0, first=False, prefetch=True)
        return carry

    lax.fori_loop(0, (_BASE_BLOCKS - 3) // 2, _pair_step, 0)
    _block(_BASE_BLOCKS - 2, 1, first=False, prefetch=True)
    _block(_BASE_BLOCKS - 1, 0, first=False, prefetch=False)
    _wait_scatter(0)
    plsc.subcore_barrier()

    @pl.when(s < _NS - 1)
    def _():
        pltpu.sync_copy(acc.at[pl.ds(zbase, _ROWS_PER_TILE)],
                        out_hbm.at[c, pl.ds(zbase, _ROWS_PER_TILE)])

    @pl.when(s == _NS - 1)
    def _():
        last = (_NS - 1) * _ROWS_PER_TILE
        pltpu.sync_copy(acc.at[pl.ds(last, _N - last)],
                        out_hbm.at[c, pl.ds(last, _N - last)])


@functools.cache
def _sc_edge_kernel():
    return pl.kernel(
        _sc_edge_body,
        out_type=jax.ShapeDtypeStruct((_NC, _N, _F), jnp.float32),
        mesh=plsc.VectorSubcoreMesh(core_axis_name="c", subcore_axis_name="s",
                                    num_cores=_NC, num_subcores=_NS),
        scratch_types=[
            pltpu.VMEM((2, _EB), jnp.int32),
            pltpu.VMEM((2, _EB), jnp.int32),
            pltpu.VMEM((2, _EB, _F), jnp.float32),
            pltpu.VMEM((2, _EB, _F), jnp.float32),
            pltpu.VMEM((16, _F), jnp.float32),
            pltpu.VMEM_SHARED((_N, _F), jnp.float32),
            pltpu.SemaphoreType.DMA((2,)),
            pltpu.SemaphoreType.DMA((2,)),
            pltpu.SemaphoreType.DMA((2,)),
            pltpu.SemaphoreType.DMA((2,)),
            pltpu.SemaphoreType.DMA((2,)),
            pltpu.SemaphoreType.DMA,
        ],
    )


def _sc_edge(h, g, edge_index):
    return _sc_edge_kernel()(h, g, edge_index.reshape(-1))


def kernel(x, edge_index, edge_attr, u, W_same, b_same, W_diff, b_diff, W_G,
           res0_W1, res0_b1, res0_W2, res0_b2,
           res1_W1, res1_b1, res1_W2, res1_b2,
           res2_W1, res2_b1, res2_W2, res2_b2,
           W_last, b_last):
    pre_grid = 25
    node_br = _N // pre_grid
    gate_br = _E // pre_grid
    h, s, g = pl.pallas_call(
        _pre_body,
        grid=(pre_grid,),
        in_specs=[
            pl.BlockSpec((node_br, _F), lambda i: (i, 0)),
            pl.BlockSpec((_F, _F), lambda i: (0, 0)),
            pl.BlockSpec((1, _F), lambda i: (0, 0)),
            pl.BlockSpec((_F, _F), lambda i: (0, 0)),
            pl.BlockSpec((1, _F), lambda i: (0, 0)),
            pl.BlockSpec((gate_br, _K), lambda i: (i, 0)),
            pl.BlockSpec((_F, _K), lambda i: (0, 0)),
        ],
        out_specs=[
            pl.BlockSpec((node_br, _F), lambda i: (i, 0)),
            pl.BlockSpec((node_br, _F), lambda i: (i, 0)),
            pl.BlockSpec((gate_br, _F), lambda i: (i, 0)),
        ],
        out_shape=[
            jax.ShapeDtypeStruct((_N, _F), jnp.float32),
            jax.ShapeDtypeStruct((_N, _F), jnp.float32),
            jax.ShapeDtypeStruct((_E, _F), jnp.float32),
        ],
    )(x, W_diff, b_diff[None, :], W_same, b_same[None, :], edge_attr, W_G)

    partials = _sc_edge(h, g, edge_index)

    mlp_br = 2000
    row_spec = pl.BlockSpec((mlp_br, _F), lambda i: (i, 0))
    p_spec = pl.BlockSpec((_NC, mlp_br, _F), lambda i: (0, i, 0))
    w_spec = pl.BlockSpec((_F, _F), lambda i: (0, 0))
    b_spec = pl.BlockSpec((1, _F), lambda i: (0, 0))
    out1, out2 = pl.pallas_call(
        _mlp_body,
        grid=(_N // mlp_br,),
        in_specs=[p_spec, row_spec, row_spec, b_spec,
                  w_spec, b_spec, w_spec, b_spec,
                  w_spec, b_spec, w_spec, b_spec,
                  w_spec, b_spec, w_spec, b_spec,
                  w_spec, b_spec],
        out_specs=[row_spec, row_spec],
        out_shape=[
            jax.ShapeDtypeStruct((_N, _F), jnp.float32),
            jax.ShapeDtypeStruct((_N, _F), jnp.float32),
        ],
    )(partials, s, x, u,
      res0_W1, res0_b1[None, :], res0_W2, res0_b2[None, :],
      res1_W1, res1_b1[None, :], res1_W2, res1_b2[None, :],
      res2_W1, res2_b1[None, :], res2_W2, res2_b2[None, :],
      W_last, b_last[None, :])
    return (out1, out2)
